# Initial kernel scaffold; baseline (speedup 1.0000x reference)
import jax
import jax.numpy as jnp
from jax.experimental import pallas as pl


def kernel(x, offset, W, b):
    raise NotImplementedError("write your pallas kernel here")


# speedup vs baseline: 1364.3022x; 1364.3022x over previous
import functools

import numpy as np
import jax
import jax.numpy as jnp
from jax import lax
from jax.experimental import pallas as pl
from jax.experimental.pallas import tpu as pltpu
from jax.experimental.pallas import tpu_sc as plsc

KH = KW = 3
H = Wd = 224
C = 96
NF = 96
NPTS = KH * KW
P = H * Wd
KC = NPTS * C

_NC, _NS = 2, 16
_NWORK = _NC * _NS
_BLK = 32
_NBLK = P // (_NWORK * _BLK)
_LG = C // 16

_INITIAL = np.stack(np.meshgrid(np.arange(KH), np.arange(KW),
                                indexing="ij")).reshape(-1, 2)


def _prep_body(off_ref, idx_ref, wts_ref):
    ri = lax.broadcasted_iota(jnp.int32, (H, Wd), 0).astype(jnp.float32)
    ci = lax.broadcasted_iota(jnp.int32, (H, Wd), 1).astype(jnp.float32)
    for k in range(NPTS):
        ky, kx = int(_INITIAL[k, 0]), int(_INITIAL[k, 1])
        y = jnp.clip(ri + float(ky - 1) + off_ref[2 * k], 0.0, float(H - 1))
        x = jnp.clip(ci + float(kx - 1) + off_ref[2 * k + 1], 0.0, float(Wd - 1))
        y0f = jnp.floor(y)
        x0f = jnp.floor(x)
        fy = y - y0f
        fx = x - x0f
        y0 = y0f.astype(jnp.int32)
        x0 = x0f.astype(jnp.int32)
        y1 = y0 + (fy > 0.0).astype(jnp.int32)
        x1 = x0 + (fx > 0.0).astype(jnp.int32)
        b0 = y0 * Wd
        b1 = y1 * Wd
        idx_ref[0, k] = b0 + x0
        idx_ref[1, k] = b1 + x0
        idx_ref[2, k] = b0 + x1
        idx_ref[3, k] = b1 + x1
        gy = 1.0 - fy
        gx = 1.0 - fx
        wts_ref[0, k] = gy * gx
        wts_ref[1, k] = fy * gx
        wts_ref[2, k] = gy * fx
        wts_ref[3, k] = fy * fx


def _prep(off_t):
    return pl.pallas_call(
        _prep_body,
        out_shape=(
            jax.ShapeDtypeStruct((4, NPTS, H, Wd), jnp.int32),
            jax.ShapeDtypeStruct((4, NPTS, H, Wd), jnp.float32),
        ),
    )(off_t)


def _splat16(vec, ids):
    return lax.gather(
        vec, ids[:, None],
        dimension_numbers=lax.GatherDimensionNumbers(
            offset_dims=(), collapsed_slice_dims=(0,), start_index_map=(0,)),
        slice_sizes=(1,),
        mode=lax.GatherScatterMode.PROMISE_IN_BOUNDS)


def _sc_body(x_hbm, idx_hbm, wts_hbm, map_hbm, idxv, wtsv, rows, mapv, sem):
    cid = lax.axis_index("c")
    sid = lax.axis_index("s")
    wid = sid * _NC + cid

    def block_body(j, carry):
        base = (wid * _NBLK + j) * _BLK

        def k_body(k, carry2):
            for cc in range(4):
                pltpu.sync_copy(idx_hbm.at[cc, k, pl.ds(base, _BLK)],
                                idxv.at[cc])
                pltpu.sync_copy(wts_hbm.at[cc, k, pl.ds(base, _BLK)],
                                wtsv.at[cc])
            cps = [pltpu.async_copy(x_hbm.at[idxv.at[cc]], rows.at[cc], sem)
                   for cc in range(4)]
            for cp in cps:
                cp.wait()
            col0 = k * C
            for g in range(_BLK // 16):
                wvec = [wtsv[cc, pl.ds(g * 16, 16)] for cc in range(4)]
                for u in range(16):
                    i = g * 16 + u
                    lane = jnp.full((16,), u, jnp.int32)
                    ws = [_splat16(wvec[cc], lane) for cc in range(4)]
                    for h in range(_LG):
                        acc = ws[0] * rows[0, i, pl.ds(h * 16, 16)]
                        acc += ws[1] * rows[1, i, pl.ds(h * 16, 16)]
                        acc += ws[2] * rows[2, i, pl.ds(h * 16, 16)]
                        acc += ws[3] * rows[3, i, pl.ds(h * 16, 16)]
                        mapv[i, pl.ds(col0 + h * 16, 16)] = acc
            return carry2

        lax.fori_loop(0, NPTS, k_body, 0)
        pltpu.sync_copy(mapv, map_hbm.at[pl.ds(base, _BLK)])
        return carry

    lax.fori_loop(0, _NBLK, block_body, 0)


def _sc_gather(x2, idx, wts):
    mesh = plsc.VectorSubcoreMesh(core_axis_name="c", subcore_axis_name="s")
    fn = pl.kernel(
        _sc_body,
        out_type=jax.ShapeDtypeStruct((P, KC), jnp.float32),
        mesh=mesh,
        scratch_types=[
            pltpu.VMEM((4, _BLK), jnp.int32),
            pltpu.VMEM((4, _BLK), jnp.float32),
            pltpu.VMEM((4, _BLK, C), jnp.float32),
            pltpu.VMEM((_BLK, KC), jnp.float32),
            pltpu.SemaphoreType.DMA,
        ],
        compiler_params=pltpu.CompilerParams(use_tc_tiling_on_sc=False),
    )
    return fn(x2, idx, wts)


_BM = 512


def _mm_body(a_ref, w_ref, b_ref, o_ref):
    o_ref[...] = jnp.dot(a_ref[...], w_ref[...],
                         preferred_element_type=jnp.float32,
                         precision=lax.Precision.HIGHEST) + b_ref[...]


def _matmul(mapped, w2, b2):
    return pl.pallas_call(
        _mm_body,
        grid=(P // _BM,),
        in_specs=[
            pl.BlockSpec((_BM, KC), lambda i: (i, 0)),
            pl.BlockSpec((KC, NF), lambda i: (0, 0)),
            pl.BlockSpec((1, NF), lambda i: (0, 0)),
        ],
        out_specs=pl.BlockSpec((_BM, NF), lambda i: (i, 0)),
        out_shape=jax.ShapeDtypeStruct((P, NF), jnp.float32),
    )(mapped, w2, b2)


def kernel(x, offset, W, b):
    off_t = offset.reshape(H, Wd, 2 * NPTS).transpose(2, 0, 1)
    idx4, wts4 = _prep(off_t)
    idx = idx4.reshape(4, NPTS, P)
    wts = wts4.reshape(4, NPTS, P)
    mapped = _sc_gather(x.reshape(P, C), idx, wts)
    out2 = _matmul(mapped, W.reshape(KC, NF), b.reshape(1, NF))
    return out2.reshape(1, H, Wd, NF)


# speedup vs baseline: 2088.2398x; 1.5306x over previous
import functools

import numpy as np
import jax
import jax.numpy as jnp
from jax import lax
from jax.experimental import pallas as pl
from jax.experimental.pallas import tpu as pltpu
from jax.experimental.pallas import tpu_sc as plsc

KH = KW = 3
H = Wd = 224
C = 96
NF = 96
NPTS = KH * KW
P = H * Wd
KC = NPTS * C

_NC, _NS = 2, 16
_NWORK = _NC * _NS
_BLK = 32
_NBLK = P // (_NWORK * _BLK)
_LG = C // 16

_INITIAL = np.stack(np.meshgrid(np.arange(KH), np.arange(KW),
                                indexing="ij")).reshape(-1, 2)


def _prep_body(off_ref, idx_ref, wts_ref):
    ri = lax.broadcasted_iota(jnp.int32, (H, Wd), 0).astype(jnp.float32)
    ci = lax.broadcasted_iota(jnp.int32, (H, Wd), 1).astype(jnp.float32)
    for k in range(NPTS):
        ky, kx = int(_INITIAL[k, 0]), int(_INITIAL[k, 1])
        y = jnp.clip(ri + float(ky - 1) + off_ref[2 * k], 0.0, float(H - 1))
        x = jnp.clip(ci + float(kx - 1) + off_ref[2 * k + 1], 0.0, float(Wd - 1))
        y0f = jnp.floor(y)
        x0f = jnp.floor(x)
        fy = y - y0f
        fx = x - x0f
        y0 = y0f.astype(jnp.int32)
        x0 = x0f.astype(jnp.int32)
        y1 = y0 + (fy > 0.0).astype(jnp.int32)
        x1 = x0 + (fx > 0.0).astype(jnp.int32)
        b0 = y0 * Wd
        b1 = y1 * Wd
        idx_ref[0, k] = b0 + x0
        idx_ref[1, k] = b1 + x0
        idx_ref[2, k] = b0 + x1
        idx_ref[3, k] = b1 + x1
        gy = 1.0 - fy
        gx = 1.0 - fx
        wts_ref[0, k] = gy * gx
        wts_ref[1, k] = fy * gx
        wts_ref[2, k] = gy * fx
        wts_ref[3, k] = fy * fx


def _prep(off_t):
    return pl.pallas_call(
        _prep_body,
        out_shape=(
            jax.ShapeDtypeStruct((4, NPTS, H, Wd), jnp.int32),
            jax.ShapeDtypeStruct((4, NPTS, H, Wd), jnp.float32),
        ),
    )(off_t)


def _splat16(vec, ids):
    return lax.gather(
        vec, ids[:, None],
        dimension_numbers=lax.GatherDimensionNumbers(
            offset_dims=(), collapsed_slice_dims=(0,), start_index_map=(0,)),
        slice_sizes=(1,),
        mode=lax.GatherScatterMode.PROMISE_IN_BOUNDS)


def _sc_body(x_hbm, idx_hbm, wts_hbm, map_hbm, idxv, wtsv, rows, mapv, sems):
    cid = lax.axis_index("c")
    sid = lax.axis_index("s")
    wid = sid * _NC + cid

    def fire(k, par):
        for cc in range(4):
            pltpu.async_copy(x_hbm.at[idxv.at[cc, k]], rows.at[par, cc],
                             sems[par])

    def drain(par):
        for cc in range(4):
            pltpu.make_async_copy(x_hbm.at[idxv.at[cc, 0]],
                                  rows.at[par, cc], sems[par]).wait()

    def combine(k, par):
        col0 = k * C
        for g in range(_BLK // 16):
            wvec = [wtsv[cc, k, pl.ds(g * 16, 16)] for cc in range(4)]
            for u in range(16):
                i = g * 16 + u
                lane = jnp.full((16,), u, jnp.int32)
                ws = [_splat16(wvec[cc], lane) for cc in range(4)]
                for h in range(_LG):
                    acc = ws[0] * rows[par, 0, i, pl.ds(h * 16, 16)]
                    acc += ws[1] * rows[par, 1, i, pl.ds(h * 16, 16)]
                    acc += ws[2] * rows[par, 2, i, pl.ds(h * 16, 16)]
                    acc += ws[3] * rows[par, 3, i, pl.ds(h * 16, 16)]
                    mapv[i, pl.ds(col0 + h * 16, 16)] = acc

    def block_body(j, carry):
        base = (wid * _NBLK + j) * _BLK
        pltpu.sync_copy(idx_hbm.at[:, :, pl.ds(base, _BLK)], idxv)
        pltpu.sync_copy(wts_hbm.at[:, :, pl.ds(base, _BLK)], wtsv)
        fire(0, 0)

        def kk_body(kk, carry2):
            k0 = 2 * kk
            fire(k0 + 1, 1)
            drain(0)
            combine(k0, 0)
            fire(k0 + 2, 0)
            drain(1)
            combine(k0 + 1, 1)
            return carry2

        lax.fori_loop(0, (NPTS - 1) // 2, kk_body, 0)
        drain(0)
        combine(NPTS - 1, 0)
        pltpu.sync_copy(mapv, map_hbm.at[pl.ds(base, _BLK)])
        return carry

    lax.fori_loop(0, _NBLK, block_body, 0)


def _sc_gather(x2, idx, wts):
    mesh = plsc.VectorSubcoreMesh(core_axis_name="c", subcore_axis_name="s")
    fn = pl.kernel(
        _sc_body,
        out_type=jax.ShapeDtypeStruct((P, KC), jnp.float32),
        mesh=mesh,
        scratch_types=[
            pltpu.VMEM((4, NPTS, _BLK), jnp.int32),
            pltpu.VMEM((4, NPTS, _BLK), jnp.float32),
            pltpu.VMEM((2, 4, _BLK, C), jnp.float32),
            pltpu.VMEM((_BLK, KC), jnp.float32),
            [pltpu.SemaphoreType.DMA, pltpu.SemaphoreType.DMA],
        ],
        compiler_params=pltpu.CompilerParams(use_tc_tiling_on_sc=False),
    )
    return fn(x2, idx, wts)


_BM = 512


def _mm_body(a_ref, w_ref, b_ref, o_ref):
    o_ref[...] = jnp.dot(a_ref[...], w_ref[...],
                         preferred_element_type=jnp.float32,
                         precision=lax.Precision.HIGHEST) + b_ref[...]


def _matmul(mapped, w2, b2):
    return pl.pallas_call(
        _mm_body,
        grid=(P // _BM,),
        in_specs=[
            pl.BlockSpec((_BM, KC), lambda i: (i, 0)),
            pl.BlockSpec((KC, NF), lambda i: (0, 0)),
            pl.BlockSpec((1, NF), lambda i: (0, 0)),
        ],
        out_specs=pl.BlockSpec((_BM, NF), lambda i: (i, 0)),
        out_shape=jax.ShapeDtypeStruct((P, NF), jnp.float32),
    )(mapped, w2, b2)


def kernel(x, offset, W, b):
    off_t = offset.reshape(H, Wd, 2 * NPTS).transpose(2, 0, 1)
    idx4, wts4 = _prep(off_t)
    idx = idx4.reshape(4, NPTS, P)
    wts = wts4.reshape(4, NPTS, P)
    mapped = _sc_gather(x.reshape(P, C), idx, wts)
    out2 = _matmul(mapped, W.reshape(KC, NF), b.reshape(1, NF))
    return out2.reshape(1, H, Wd, NF)
